# baseline (speedup 1.0000x reference)
import jax
import jax.numpy as jnp
from jax import lax
from jax.experimental import pallas as pl
from jax.experimental.pallas import tpu as pltpu

N = 4096
D1, D2, D3, D_IN = 128, 256, 512, 512
PB = 512
CACHE_PANELS = 4
TMI = 512
TMJ = 1024


def _body(z_ref, adj_ref, w4_ref, w5_ref, w6_ref,
          zhat_ref, recon_ref,
          abuf_ref, acache_ref, buf1_ref, buf2_ref, zstage_ref, rbuf_ref,
          in_sem, zh_sem, out_sem):
    np_ = N // PB
    n_tail = np_ - CACHE_PANELS
    tj = N // TMJ
    bf = jnp.bfloat16

    def in_copy(k, slot):
        return pltpu.make_async_copy(
            adj_ref.at[pl.ds(k * PB, PB), :],
            abuf_ref.at[slot],
            in_sem.at[slot],
        )

    in_copy(0, 0).start()
    acc = jnp.dot(z_ref[...].astype(bf), w4_ref[...].astype(bf),
                  preferred_element_type=jnp.float32)
    buf1_ref[:, :D2] = jnp.tanh(acc).astype(bf)

    w5 = w5_ref[...].astype(bf)
    w6 = w6_ref[...].astype(bf)

    def pass_a(k, carry):
        slot = lax.rem(k, 2)

        @pl.when(k < np_ - 1)
        def _():
            in_copy(k + 1, lax.rem(k + 1, 2)).start()

        in_copy(k, slot).wait()
        a = abuf_ref[slot].astype(bf)

        @pl.when(k < CACHE_PANELS)
        def _():
            acache_ref[pl.ds(k * PB, PB), :] = a

        acc = jnp.dot(a, buf1_ref[:, :D2], preferred_element_type=jnp.float32)
        r = jnp.dot(acc.astype(bf), w5, preferred_element_type=jnp.float32)
        buf2_ref[pl.ds(k * PB, PB), :] = jnp.tanh(r).astype(bf)
        return carry

    lax.fori_loop(0, np_, pass_a, 0, unroll=False)

    def tail_copy(g, slot):
        k = CACHE_PANELS + lax.rem(g, n_tail)
        return pltpu.make_async_copy(
            adj_ref.at[pl.ds(k * PB, PB), :],
            abuf_ref.at[slot],
            in_sem.at[slot],
        )

    tail_copy(0, 0).start()
    tail_copy(1, 1).start()

    def zh_copy(m, slot):
        return pltpu.make_async_copy(
            zstage_ref.at[slot],
            zhat_ref.at[pl.ds(m * PB, PB), :],
            zh_sem.at[slot],
        )

    def compute_panel(p, k, a):
        rows = pl.ds(k * PB, PB)
        if p == 0:
            acc = jnp.dot(a, buf2_ref[...], preferred_element_type=jnp.float32)
            r = jnp.dot(acc.astype(bf), w6, preferred_element_type=jnp.float32)
            buf1_ref[rows, :] = r.astype(bf)
        else:
            acc = jnp.dot(a, buf1_ref[...], preferred_element_type=jnp.float32)
            slot = lax.rem(k, 2)

            @pl.when(k >= 2)
            def _():
                zh_copy(k - 2, slot).wait()

            zstage_ref[slot] = acc
            zh_copy(k, slot).start()
            buf2_ref[rows, :] = acc.astype(bf)

    def make_cached(p):
        def body_fn(k, carry):
            compute_panel(p, k, acache_ref[pl.ds(k * PB, PB), :])
            return carry
        return body_fn

    def make_tail(p):
        def body_fn(k, carry):
            g = p * n_tail + (k - CACHE_PANELS)
            slot = lax.rem(g, 2)
            tail_copy(g, slot).wait()
            compute_panel(p, k, abuf_ref[slot].astype(bf))

            @pl.when(g + 2 < 2 * n_tail)
            def _():
                tail_copy(g + 2, slot).start()
            return carry
        return body_fn

    for p in range(2):
        lax.fori_loop(0, CACHE_PANELS, make_cached(p), 0, unroll=False)
        lax.fori_loop(CACHE_PANELS, np_, make_tail(p), 0, unroll=False)

    zh_copy(np_ - 2, lax.rem(np_ - 2, 2)).wait()
    zh_copy(np_ - 1, lax.rem(np_ - 1, 2)).wait()

    def out_copy(t, slot):
        i = t // tj
        j = lax.rem(t, tj)
        return pltpu.make_async_copy(
            rbuf_ref.at[slot],
            recon_ref.at[pl.ds(i * TMI, TMI), pl.ds(j * TMJ, TMJ)],
            out_sem.at[slot],
        )

    n_tiles = (N // TMI) * tj

    def recon_body(t, carry):
        slot = lax.rem(t, 2)

        @pl.when(t >= 2)
        def _():
            out_copy(t - 2, slot).wait()

        i = t // tj
        j = lax.rem(t, tj)
        a = buf2_ref[pl.ds(i * TMI, TMI), :]
        b = buf2_ref[pl.ds(j * TMJ, TMJ), :]
        acc = lax.dot_general(
            a, b, dimension_numbers=(((1,), (1,)), ((), ())),
            preferred_element_type=jnp.float32)
        rbuf_ref[slot] = 0.5 * (1.0 + jnp.tanh(0.5 * acc))
        out_copy(t, slot).start()
        return carry

    lax.fori_loop(0, n_tiles, recon_body, 0, unroll=False)
    out_copy(n_tiles - 2, 0).wait()
    out_copy(n_tiles - 1, 1).wait()


def kernel(z_igae, adj, W4, W5, W6):
    z_hat, z_hat_adj = pl.pallas_call(
        _body,
        in_specs=[
            pl.BlockSpec(memory_space=pltpu.VMEM),
            pl.BlockSpec(memory_space=pl.ANY),
            pl.BlockSpec(memory_space=pltpu.VMEM),
            pl.BlockSpec(memory_space=pltpu.VMEM),
            pl.BlockSpec(memory_space=pltpu.VMEM),
        ],
        out_specs=[
            pl.BlockSpec(memory_space=pl.ANY),
            pl.BlockSpec(memory_space=pl.ANY),
        ],
        out_shape=[
            jax.ShapeDtypeStruct((N, D_IN), jnp.float32),
            jax.ShapeDtypeStruct((N, N), jnp.float32),
        ],
        scratch_shapes=[
            pltpu.VMEM((2, PB, N), jnp.float32),
            pltpu.VMEM((CACHE_PANELS * PB, N), jnp.bfloat16),
            pltpu.VMEM((N, D3), jnp.bfloat16),
            pltpu.VMEM((N, D_IN), jnp.bfloat16),
            pltpu.VMEM((2, PB, D_IN), jnp.float32),
            pltpu.VMEM((2, TMI, TMJ), jnp.float32),
            pltpu.SemaphoreType.DMA((2,)),
            pltpu.SemaphoreType.DMA((2,)),
            pltpu.SemaphoreType.DMA((2,)),
        ],
    )(z_igae, adj, W4, W5, W6)
    return (z_hat, z_hat_adj)


# speedup vs baseline: 1.0623x; 1.0623x over previous
import jax
import jax.numpy as jnp
from jax.experimental import pallas as pl
from jax.experimental.pallas import tpu as pltpu

N = 4096
D1, D2, D3, D_IN = 128, 256, 512, 512
BM = 512
TMI = 1024
TMJ = 2048


def _mega_kernel(z_ref, adj_ref, w4_ref, w5_ref, w6_ref,
                 zhat_ref, recon_ref,
                 s1_ref, s2_ref, s3_ref, zh_ref, *, np_, bm, tmi, tmj, tj):
    s = pl.program_id(0)

    @pl.when(s == 0)
    def _s1():
        z = z_ref[...].astype(jnp.bfloat16)
        w4 = w4_ref[...].astype(jnp.bfloat16)
        acc = jnp.dot(z, w4, preferred_element_type=jnp.float32)
        s1_ref[...] = jnp.tanh(acc).astype(jnp.bfloat16)

    @pl.when(s < np_)
    def _stage_a():
        a = adj_ref[...].astype(jnp.bfloat16)
        acc = jnp.dot(a, s1_ref[...], preferred_element_type=jnp.float32)
        w5 = w5_ref[...].astype(jnp.bfloat16)
        r = jnp.dot(acc.astype(jnp.bfloat16), w5,
                    preferred_element_type=jnp.float32)
        s2_ref[pl.ds(s * bm, bm), :] = jnp.tanh(r).astype(jnp.bfloat16)

    @pl.when((s >= np_) & (s < 2 * np_))
    def _stage_b():
        a = adj_ref[...].astype(jnp.bfloat16)
        acc = jnp.dot(a, s2_ref[...], preferred_element_type=jnp.float32)
        w6 = w6_ref[...].astype(jnp.bfloat16)
        r = jnp.dot(acc.astype(jnp.bfloat16), w6,
                    preferred_element_type=jnp.float32)
        s3_ref[pl.ds((s - np_) * bm, bm), :] = r.astype(jnp.bfloat16)

    @pl.when((s >= 2 * np_) & (s < 3 * np_))
    def _stage_c():
        a = adj_ref[...].astype(jnp.bfloat16)
        acc = jnp.dot(a, s3_ref[...], preferred_element_type=jnp.float32)
        zhat_ref[...] = acc
        zh_ref[pl.ds((s - 2 * np_) * bm, bm), :] = acc.astype(jnp.bfloat16)

    @pl.when(s >= 3 * np_)
    def _stage_d():
        t = s - 3 * np_
        i = t // tj
        j = t % tj
        a = zh_ref[pl.ds(i * tmi, tmi), :]
        b = zh_ref[pl.ds(j * tmj, tmj), :]
        acc = jax.lax.dot_general(
            a, b, dimension_numbers=(((1,), (1,)), ((), ())),
            preferred_element_type=jnp.float32)
        recon_ref[...] = 0.5 * (1.0 + jnp.tanh(0.5 * acc))


def kernel(z_igae, adj, W4, W5, W6):
    n = N
    bm, tmi, tmj = BM, TMI, TMJ
    np_ = n // bm
    ti = n // tmi
    tj = n // tmj
    steps = 3 * np_ + ti * tj

    def adj_map(s):
        return (jnp.minimum(s, 3 * np_ - 1) % np_, 0)

    def zhat_map(s):
        return (jnp.clip(s - 2 * np_, 0, np_ - 1), 0)

    def recon_map(s):
        t = jnp.maximum(s - 3 * np_, 0)
        return (t // tj, t % tj)

    import functools
    kern = functools.partial(_mega_kernel, np_=np_, bm=bm, tmi=tmi, tmj=tmj, tj=tj)
    z_hat, z_hat_adj = pl.pallas_call(
        kern,
        grid=(steps,),
        in_specs=[
            pl.BlockSpec((n, D1), lambda s: (0, 0)),
            pl.BlockSpec((bm, n), adj_map),
            pl.BlockSpec((D1, D2), lambda s: (0, 0)),
            pl.BlockSpec((D2, D3), lambda s: (0, 0)),
            pl.BlockSpec((D3, D_IN), lambda s: (0, 0)),
        ],
        out_specs=[
            pl.BlockSpec((bm, D_IN), zhat_map),
            pl.BlockSpec((tmi, tmj), recon_map),
        ],
        out_shape=[
            jax.ShapeDtypeStruct((n, D_IN), jnp.float32),
            jax.ShapeDtypeStruct((n, n), jnp.float32),
        ],
        scratch_shapes=[
            pltpu.VMEM((n, D2), jnp.bfloat16),
            pltpu.VMEM((n, D3), jnp.bfloat16),
            pltpu.VMEM((n, D_IN), jnp.bfloat16),
            pltpu.VMEM((n, D_IN), jnp.bfloat16),
        ],
        compiler_params=pltpu.CompilerParams(
            dimension_semantics=("arbitrary",),
        ),
    )(z_igae, adj, W4, W5, W6)
    return (z_hat, z_hat_adj)
